# baseline (speedup 1.0000x reference)
import functools

import jax
import jax.numpy as jnp
from jax import lax
from jax.experimental import pallas as pl
from jax.experimental.pallas import tpu as pltpu
from jax.experimental.pallas import tpu_sc as plsc

_N = 50000
_E = 800000
_DF = 64
_DH = 128
_NH = 25000
_NPH = 25024
_NT = 16
_KB = 128
_D = 3
_CHJ = 6
_NJB = 66
_EPT = _NJB * _CHJ * _KB
_STRIPE = _NPH // _NT
_BR = 1000
_NB = _N // _BR
_HB = _NH // _BR
_DT = 1.0 / 9.0
_LN_EPS = 1e-5


@functools.cache
def _sc_edge_scatter():
    mesh = plsc.VectorSubcoreMesh(core_axis_name="c", subcore_axis_name="s")

    @functools.partial(
        pl.kernel,
        out_type=jax.ShapeDtypeStruct((2, _NPH, _DH), jnp.bfloat16),
        mesh=mesh,
        scratch_types=[
            pltpu.VMEM((2, _CHJ, _KB), jnp.int32),
            pltpu.VMEM((2, _CHJ, _KB), jnp.int32),
            pltpu.VMEM((_D, _KB, _DH), jnp.bfloat16),
            pltpu.VMEM_SHARED((_NPH, _DH), jnp.bfloat16),
            [pltpu.SemaphoreType.DMA] * _D,
            [pltpu.SemaphoreType.DMA] * _D,
            [pltpu.SemaphoreType.DMA] * 2,
        ],
        compiler_params=pltpu.CompilerParams(use_tc_tiling_on_sc=False),
    )
    def k(yh, srcA, srcB, dstA, dstB, out, src_v, dst_v, rows_v, acc,
          gsem, ssem, isem):
        c = lax.axis_index("c")
        t = lax.axis_index("s")
        z32 = jnp.zeros((32,), jnp.bfloat16)

        def zb(j, carry):
            for o in range(0, _DH, 32):
                rows_v[0, j, pl.ds(o, 32)] = z32
            return carry

        def do_pass(srch, dsth, q):
            def gather_start(slot, b, row):
                pltpu.async_copy(
                    yh.at[src_v.at[b, row]], rows_v.at[slot], gsem[slot])

            def gather_wait(slot):
                pltpu.make_async_copy(
                    yh.at[src_v.at[0, 0]], rows_v.at[slot],
                    gsem[slot]).wait()

            def scatter_start(slot, b, row):
                pltpu.async_copy(
                    rows_v.at[slot], acc.at[dst_v.at[b, row]], ssem[slot],
                    add=True)

            def scatter_wait(slot):
                pltpu.make_async_copy(
                    rows_v.at[slot], acc.at[dst_v.at[0, 0]],
                    ssem[slot]).wait()

            lax.fori_loop(0, _KB, zb, 0)

            def zc(i, carry):
                pltpu.sync_copy(
                    rows_v.at[0],
                    acc.at[pl.ds(t * _STRIPE + i * _KB, _KB)])
                return carry

            lax.fori_loop(0, _STRIPE // _KB, zc, 0)
            rem = _STRIPE % _KB
            if rem:
                pltpu.sync_copy(
                    rows_v.at[0, pl.ds(0, rem)],
                    acc.at[pl.ds(t * _STRIPE + _STRIPE - rem, rem)])
            plsc.subcore_barrier()

            pltpu.sync_copy(srch.at[t, pl.ds(0, _CHJ)], src_v.at[0])
            pltpu.sync_copy(dsth.at[t, pl.ds(0, _CHJ)], dst_v.at[0])

            def blk(jj, carry):
                b = jnp.bitwise_and(jj, 1)
                pb = 1 - b

                @pl.when(jj > 0)
                def _():
                    pltpu.make_async_copy(
                        srch.at[t, pl.ds(0, _CHJ)], src_v.at[0],
                        isem[0]).wait()
                    pltpu.make_async_copy(
                        dsth.at[t, pl.ds(0, _CHJ)], dst_v.at[0],
                        isem[1]).wait()

                for jb in range(_CHJ):
                    slot = jb % _D
                    if jb >= _D:
                        scatter_wait(slot)
                    else:
                        @pl.when(jj > 0)
                        def _():
                            scatter_wait(slot)
                    gather_start(slot, b, jb)
                    s2 = (jb - 2) % _D
                    if jb >= 2:
                        gather_wait(s2)
                        scatter_start(s2, b, jb - 2)
                    else:
                        @pl.when(jj > 0)
                        def _():
                            gather_wait(s2)
                            scatter_start(s2, pb, jb + _CHJ - 2)
                    if jb == 3:
                        @pl.when(jj < _NJB - 1)
                        def _():
                            pltpu.async_copy(
                                srch.at[t, pl.ds((jj + 1) * _CHJ, _CHJ)],
                                src_v.at[pb], isem[0])
                            pltpu.async_copy(
                                dsth.at[t, pl.ds((jj + 1) * _CHJ, _CHJ)],
                                dst_v.at[pb], isem[1])
                return carry

            lax.fori_loop(0, _NJB, blk, 0)
            lb = (_NJB - 1) % 2
            gather_wait((_CHJ - 2) % _D)
            scatter_start((_CHJ - 2) % _D, lb, _CHJ - 2)
            gather_wait((_CHJ - 1) % _D)
            scatter_start((_CHJ - 1) % _D, lb, _CHJ - 1)
            for slot in range(_D):
                scatter_wait(slot)
            plsc.subcore_barrier()
            pltpu.sync_copy(acc.at[pl.ds(t * _STRIPE, _STRIPE)],
                            out.at[q, pl.ds(t * _STRIPE, _STRIPE)])
            plsc.subcore_barrier()

        @pl.when(c == 0)
        def _():
            do_pass(srcA, dstA, 0)

        @pl.when(c == 1)
        def _():
            do_pass(srcB, dstB, 1)

    return k


_HI = jax.lax.Precision.HIGHEST


def _dinv_from_ones(so_blk):
    deg = so_blk[0, :, 0:1].astype(jnp.float32) + 1.0
    return lax.rsqrt(jnp.maximum(deg, 1e-12))


def _s_spec():
    return pl.BlockSpec((1, _BR, _DH), lambda r: (r // _HB, r % _HB, 0))


def _tc_init_body(x_ref, win_ref, bin_ref, wg_ref, so_ref, h_ref, y_ref):
    dinv = _dinv_from_ones(so_ref[...])
    h = jnp.dot(x_ref[...], win_ref[...], precision=_HI,
                preferred_element_type=jnp.float32) + bin_ref[...]
    h_ref[...] = h
    y = dinv * jnp.dot(h, wg_ref[...], precision=_HI,
                       preferred_element_type=jnp.float32)
    y_ref[...] = y.astype(jnp.bfloat16)


@functools.cache
def _tc_init():
    row = lambda r: (r, 0)
    fixed = lambda r: (0, 0)
    return pl.pallas_call(
        _tc_init_body,
        grid=(_NB,),
        in_specs=[
            pl.BlockSpec((_BR, _DF), row),
            pl.BlockSpec((_DF, _DH), fixed),
            pl.BlockSpec((1, _DH), fixed),
            pl.BlockSpec((_DH, _DH), fixed),
            _s_spec(),
        ],
        out_specs=[
            pl.BlockSpec((_BR, _DH), row),
            pl.BlockSpec((_BR, _DH), row),
        ],
        out_shape=[
            jax.ShapeDtypeStruct((_N, _DH), jnp.float32),
            jax.ShapeDtypeStruct((_N, _DH), jnp.bfloat16),
        ],
    )


def _tc_step_body(last, h_ref, y_ref, s_ref, so_ref,
                  wg_ref, bg_ref, g_ref, b_ref, *outs):
    dinv = _dinv_from_ones(so_ref[...])
    z = s_ref[0].astype(jnp.float32) + y_ref[...].astype(jnp.float32)
    pre = dinv * z + bg_ref[...]
    mu = jnp.mean(pre, axis=-1, keepdims=True)
    d = pre - mu
    var = jnp.mean(d * d, axis=-1, keepdims=True)
    dh = jnp.tanh(d * lax.rsqrt(var + _LN_EPS) * g_ref[...] + b_ref[...])
    h_new = h_ref[...] + _DT * dh
    outs[0][...] = h_new
    if not last:
        y = dinv * jnp.dot(h_new, wg_ref[...], precision=_HI,
                           preferred_element_type=jnp.float32)
        outs[1][...] = y.astype(jnp.bfloat16)


@functools.cache
def _tc_step(last):
    row = lambda r: (r, 0)
    fixed = lambda r: (0, 0)
    n_y_out = 0 if last else 1
    return pl.pallas_call(
        functools.partial(_tc_step_body, last),
        grid=(_NB,),
        in_specs=[
            pl.BlockSpec((_BR, _DH), row),
            pl.BlockSpec((_BR, _DH), row),
            _s_spec(),
            _s_spec(),
            pl.BlockSpec((_DH, _DH), fixed),
            pl.BlockSpec((1, _DH), fixed),
            pl.BlockSpec((1, _DH), fixed),
            pl.BlockSpec((1, _DH), fixed),
        ],
        out_specs=[pl.BlockSpec((_BR, _DH), row)] +
                  [pl.BlockSpec((_BR, _DH), row)] * n_y_out,
        out_shape=[jax.ShapeDtypeStruct((_N, _DH), jnp.float32)] +
                  [jax.ShapeDtypeStruct((_N, _DH), jnp.bfloat16)] * n_y_out,
    )


def _tc_final_body(h_ref, wout_ref, bout_ref, out_ref, acc_ref):
    r = pl.program_id(0)

    @pl.when(r == 0)
    def _():
        acc_ref[...] = jnp.zeros((8, _DH), jnp.float32)

    part = jnp.sum(h_ref[...], axis=0, keepdims=True)
    acc_ref[...] = acc_ref[...] + jnp.broadcast_to(part, (8, _DH))

    @pl.when(r == _NB - 1)
    def _():
        m = acc_ref[...] * (1.0 / _N)
        out_ref[...] = jnp.dot(m, wout_ref[...], precision=_HI,
                               preferred_element_type=jnp.float32) + \
            bout_ref[...]


@functools.cache
def _tc_final():
    fixed = lambda r: (0, 0)
    return pl.pallas_call(
        _tc_final_body,
        grid=(_NB,),
        in_specs=[
            pl.BlockSpec((_BR, _DH), lambda r: (r, 0)),
            pl.BlockSpec((_DH, _DH), fixed),
            pl.BlockSpec((1, _DH), fixed),
        ],
        out_specs=pl.BlockSpec((8, _DH), fixed),
        out_shape=jax.ShapeDtypeStruct((8, _DH), jnp.float32),
        scratch_shapes=[pltpu.VMEM((8, _DH), jnp.float32)],
    )


def kernel(x, edge_index, W_in, b_in, W_gcn, b_gcn, ln_g, ln_b, W_out, b_out):
    src = edge_index[0]
    dst = edge_index[1]
    pad = _NT * _EPT - _E
    srcp = jnp.concatenate([src, jnp.zeros((pad,), src.dtype)])
    dstp = jnp.concatenate([dst, jnp.full((pad,), _N, dst.dtype)])
    in_a = dstp < _NH
    shape = (_NT, _NJB * _CHJ, _KB)
    srcA = jnp.where(in_a, srcp, 0).reshape(shape)
    srcB = jnp.where(in_a, 0, srcp).reshape(shape)
    dstA = jnp.where(in_a, dstp, _NPH - 1).reshape(shape)
    dstB = jnp.where(in_a, _NPH - 1, dstp - _NH).reshape(shape)

    sc = _sc_edge_scatter()
    ones_tab = jnp.ones((_N, _DH), jnp.bfloat16)
    s_ones = sc(ones_tab, srcA, srcB, dstA, dstB)

    b_in2 = b_in.reshape(1, _DH)
    b_gcn2 = b_gcn.reshape(1, _DH)
    ln_g2 = ln_g.reshape(1, _DH)
    ln_b2 = ln_b.reshape(1, _DH)
    b_out2 = b_out.reshape(1, _DH)

    h, y = _tc_init()(x, W_in, b_in2, W_gcn, s_ones)
    for i in range(1, 10):
        s = sc(y, srcA, srcB, dstA, dstB)
        last = i == 9
        outs = _tc_step(last)(h, y, s, s_ones, W_gcn, b_gcn2, ln_g2, ln_b2)
        if last:
            (h,) = outs
        else:
            h, y = outs

    res = _tc_final()(h, W_out, b_out2)
    return res[0:1]


# speedup vs baseline: 1.0001x; 1.0001x over previous
import functools

import jax
import jax.numpy as jnp
from jax import lax
from jax.experimental import pallas as pl
from jax.experimental.pallas import tpu as pltpu
from jax.experimental.pallas import tpu_sc as plsc

_N = 50000
_E = 800000
_DF = 64
_DH = 128
_NH = 25000
_NPH = 25024
_NT = 16
_KB = 128
_D = 3
_CHJ = 6
_NJB = 66
_EPT = _NJB * _CHJ * _KB
_STRIPE = _NPH // _NT
_BR = 1000
_NB = _N // _BR
_HB = _NH // _BR
_DT = 1.0 / 9.0
_LN_EPS = 1e-5


@functools.cache
def _sc_edge_scatter():
    mesh = plsc.VectorSubcoreMesh(core_axis_name="c", subcore_axis_name="s")

    @functools.partial(
        pl.kernel,
        out_type=jax.ShapeDtypeStruct((2, _NPH, _DH), jnp.bfloat16),
        mesh=mesh,
        scratch_types=[
            pltpu.VMEM((2, _CHJ, _KB), jnp.int32),
            pltpu.VMEM((2, _CHJ, _KB), jnp.int32),
            pltpu.VMEM((_D, _KB, _DH), jnp.bfloat16),
            pltpu.VMEM_SHARED((_NPH, _DH), jnp.bfloat16),
            [pltpu.SemaphoreType.DMA] * _D,
            [pltpu.SemaphoreType.DMA] * _D,
            [pltpu.SemaphoreType.DMA] * 2,
        ],
        compiler_params=pltpu.CompilerParams(use_tc_tiling_on_sc=False),
    )
    def k(yh, srcA, srcB, dstA, dstB, out, src_v, dst_v, rows_v, acc,
          gsem, ssem, isem):
        c = lax.axis_index("c")
        t = lax.axis_index("s")
        z32 = jnp.zeros((32,), jnp.bfloat16)

        def zb(j, carry):
            for o in range(0, _DH, 32):
                rows_v[0, j, pl.ds(o, 32)] = z32
            return carry

        def do_pass(srch, dsth, q):
            def gather_start(slot, b, row):
                pltpu.async_copy(
                    yh.at[src_v.at[b, row]], rows_v.at[slot], gsem[slot])

            def gather_wait(slot):
                pltpu.make_async_copy(
                    yh.at[src_v.at[0, 0]], rows_v.at[slot],
                    gsem[slot]).wait()

            def scatter_start(slot, b, row):
                pltpu.async_copy(
                    rows_v.at[slot], acc.at[dst_v.at[b, row]], ssem[slot],
                    add=True)

            def scatter_wait(slot):
                pltpu.make_async_copy(
                    rows_v.at[slot], acc.at[dst_v.at[0, 0]],
                    ssem[slot]).wait()

            lax.fori_loop(0, _KB, zb, 0)

            def zc(i, carry):
                pltpu.sync_copy(
                    rows_v.at[0],
                    acc.at[pl.ds(t * _STRIPE + i * _KB, _KB)])
                return carry

            lax.fori_loop(0, _STRIPE // _KB, zc, 0)
            rem = _STRIPE % _KB
            if rem:
                pltpu.sync_copy(
                    rows_v.at[0, pl.ds(0, rem)],
                    acc.at[pl.ds(t * _STRIPE + _STRIPE - rem, rem)])
            plsc.subcore_barrier()

            pltpu.sync_copy(srch.at[t, pl.ds(0, _CHJ)], src_v.at[0])
            pltpu.sync_copy(dsth.at[t, pl.ds(0, _CHJ)], dst_v.at[0])

            def blk(jj, carry):
                b = jnp.bitwise_and(jj, 1)
                pb = 1 - b

                @pl.when(jj > 0)
                def _():
                    pltpu.make_async_copy(
                        srch.at[t, pl.ds(0, _CHJ)], src_v.at[0],
                        isem[0]).wait()
                    pltpu.make_async_copy(
                        dsth.at[t, pl.ds(0, _CHJ)], dst_v.at[0],
                        isem[1]).wait()

                for jb in range(_CHJ):
                    slot = jb % _D
                    if jb >= _D:
                        scatter_wait(slot)
                    else:
                        @pl.when(jj > 0)
                        def _():
                            scatter_wait(slot)
                    gather_start(slot, b, jb)
                    s2 = (jb - 2) % _D
                    if jb >= 2:
                        gather_wait(s2)
                        scatter_start(s2, b, jb - 2)
                    else:
                        @pl.when(jj > 0)
                        def _():
                            gather_wait(s2)
                            scatter_start(s2, pb, jb + _CHJ - 2)
                    if jb == 3:
                        @pl.when(jj < _NJB - 1)
                        def _():
                            pltpu.async_copy(
                                srch.at[t, pl.ds((jj + 1) * _CHJ, _CHJ)],
                                src_v.at[pb], isem[0])
                            pltpu.async_copy(
                                dsth.at[t, pl.ds((jj + 1) * _CHJ, _CHJ)],
                                dst_v.at[pb], isem[1])
                return carry

            lax.fori_loop(0, _NJB, blk, 0)
            lb = (_NJB - 1) % 2
            gather_wait((_CHJ - 2) % _D)
            scatter_start((_CHJ - 2) % _D, lb, _CHJ - 2)
            gather_wait((_CHJ - 1) % _D)
            scatter_start((_CHJ - 1) % _D, lb, _CHJ - 1)
            for slot in range(_D):
                scatter_wait(slot)
            plsc.subcore_barrier()
            pltpu.sync_copy(acc.at[pl.ds(t * _STRIPE, _STRIPE)],
                            out.at[q, pl.ds(t * _STRIPE, _STRIPE)])
            plsc.subcore_barrier()

        @pl.when(c == 0)
        def _():
            do_pass(srcA, dstA, 0)

        @pl.when(c == 1)
        def _():
            do_pass(srcB, dstB, 1)

    return k


_HI = jax.lax.Precision.HIGHEST


def _dinv_from_ones(so_blk):
    deg = so_blk[0, :, 0:1].astype(jnp.float32) + 1.0
    return lax.rsqrt(jnp.maximum(deg, 1e-12))


def _s_spec():
    return pl.BlockSpec((1, _BR, _DH), lambda r: (r // _HB, r % _HB, 0))


def _tc_init_body(x_ref, win_ref, bin_ref, wg_ref, so_ref, h_ref, y_ref):
    dinv = _dinv_from_ones(so_ref[...])
    h = jnp.dot(x_ref[...], win_ref[...], precision=_HI,
                preferred_element_type=jnp.float32) + bin_ref[...]
    h_ref[...] = h
    y = dinv * jnp.dot(h, wg_ref[...], precision=_HI,
                       preferred_element_type=jnp.float32)
    y_ref[...] = y.astype(jnp.bfloat16)


@functools.cache
def _tc_init():
    row = lambda r: (r, 0)
    fixed = lambda r: (0, 0)
    return pl.pallas_call(
        _tc_init_body,
        grid=(_NB,),
        in_specs=[
            pl.BlockSpec((_BR, _DF), row),
            pl.BlockSpec((_DF, _DH), fixed),
            pl.BlockSpec((1, _DH), fixed),
            pl.BlockSpec((_DH, _DH), fixed),
            _s_spec(),
        ],
        out_specs=[
            pl.BlockSpec((_BR, _DH), row),
            pl.BlockSpec((_BR, _DH), row),
        ],
        out_shape=[
            jax.ShapeDtypeStruct((_N, _DH), jnp.float32),
            jax.ShapeDtypeStruct((_N, _DH), jnp.bfloat16),
        ],
    )


def _tc_step_body(last, h_ref, y_ref, s_ref, so_ref,
                  wg_ref, bg_ref, g_ref, b_ref, *outs):
    dinv = _dinv_from_ones(so_ref[...])
    z = s_ref[0].astype(jnp.float32) + y_ref[...].astype(jnp.float32)
    pre = dinv * z + bg_ref[...]
    mu = jnp.mean(pre, axis=-1, keepdims=True)
    d = pre - mu
    var = jnp.mean(d * d, axis=-1, keepdims=True)
    dh = jnp.tanh(d * lax.rsqrt(var + _LN_EPS) * g_ref[...] + b_ref[...])
    h_new = h_ref[...] + _DT * dh
    outs[0][...] = h_new
    if not last:
        y = dinv * jnp.dot(h_new, wg_ref[...], precision=_HI,
                           preferred_element_type=jnp.float32)
        outs[1][...] = y.astype(jnp.bfloat16)


@functools.cache
def _tc_step(last):
    row = lambda r: (r, 0)
    fixed = lambda r: (0, 0)
    n_y_out = 0 if last else 1
    return pl.pallas_call(
        functools.partial(_tc_step_body, last),
        grid=(_NB,),
        in_specs=[
            pl.BlockSpec((_BR, _DH), row),
            pl.BlockSpec((_BR, _DH), row),
            _s_spec(),
            _s_spec(),
            pl.BlockSpec((_DH, _DH), fixed),
            pl.BlockSpec((1, _DH), fixed),
            pl.BlockSpec((1, _DH), fixed),
            pl.BlockSpec((1, _DH), fixed),
        ],
        out_specs=[pl.BlockSpec((_BR, _DH), row)] +
                  [pl.BlockSpec((_BR, _DH), row)] * n_y_out,
        out_shape=[jax.ShapeDtypeStruct((_N, _DH), jnp.float32)] +
                  [jax.ShapeDtypeStruct((_N, _DH), jnp.bfloat16)] * n_y_out,
    )


def _tc_final_body(h_ref, wout_ref, bout_ref, out_ref, acc_ref):
    r = pl.program_id(0)

    @pl.when(r == 0)
    def _():
        acc_ref[...] = jnp.zeros((8, _DH), jnp.float32)

    part = jnp.sum(h_ref[...], axis=0, keepdims=True)
    acc_ref[...] = acc_ref[...] + jnp.broadcast_to(part, (8, _DH))

    @pl.when(r == _NB - 1)
    def _():
        m = acc_ref[...] * (1.0 / _N)
        out_ref[...] = jnp.dot(m, wout_ref[...], precision=_HI,
                               preferred_element_type=jnp.float32) + \
            bout_ref[...]


@functools.cache
def _tc_final():
    fixed = lambda r: (0, 0)
    return pl.pallas_call(
        _tc_final_body,
        grid=(_NB,),
        in_specs=[
            pl.BlockSpec((_BR, _DH), lambda r: (r, 0)),
            pl.BlockSpec((_DH, _DH), fixed),
            pl.BlockSpec((1, _DH), fixed),
        ],
        out_specs=pl.BlockSpec((8, _DH), fixed),
        out_shape=jax.ShapeDtypeStruct((8, _DH), jnp.float32),
        scratch_shapes=[pltpu.VMEM((8, _DH), jnp.float32)],
    )


def kernel(x, edge_index, W_in, b_in, W_gcn, b_gcn, ln_g, ln_b, W_out, b_out):
    src = edge_index[0]
    dst = edge_index[1]
    pad = _NT * _EPT - _E
    srcp = jnp.concatenate([src, jnp.zeros((pad,), src.dtype)])
    dstp = jnp.concatenate([dst, jnp.full((pad,), _N, dst.dtype)])
    in_a = dstp < _NH
    shape = (_NT, _NJB * _CHJ, _KB)
    dummy = _NH + (jnp.arange(dstp.shape[0], dtype=dstp.dtype) % (_NPH - _NH))
    srcA = jnp.where(in_a, srcp, 0).reshape(shape)
    srcB = jnp.where(in_a, 0, srcp).reshape(shape)
    dstA = jnp.where(in_a, dstp, dummy).reshape(shape)
    dstB = jnp.where(in_a, dummy, dstp - _NH).reshape(shape)

    sc = _sc_edge_scatter()
    ones_tab = jnp.ones((_N, _DH), jnp.bfloat16)
    s_ones = sc(ones_tab, srcA, srcB, dstA, dstB)

    b_in2 = b_in.reshape(1, _DH)
    b_gcn2 = b_gcn.reshape(1, _DH)
    ln_g2 = ln_g.reshape(1, _DH)
    ln_b2 = ln_b.reshape(1, _DH)
    b_out2 = b_out.reshape(1, _DH)

    h, y = _tc_init()(x, W_in, b_in2, W_gcn, s_ones)
    for i in range(1, 10):
        s = sc(y, srcA, srcB, dstA, dstB)
        last = i == 9
        outs = _tc_step(last)(h, y, s, s_ones, W_gcn, b_gcn2, ln_g2, ln_b2)
        if last:
            (h,) = outs
        else:
            h, y = outs

    res = _tc_final()(h, W_out, b_out2)
    return res[0:1]


# speedup vs baseline: 18.7805x; 18.7793x over previous
import functools

import jax
import jax.numpy as jnp
from jax import lax
from jax.experimental import pallas as pl
from jax.experimental.pallas import tpu as pltpu
from jax.experimental.pallas import tpu_sc as plsc

_N = 50000
_E = 800000
_DF = 64
_DH = 128
_NH = 25000
_NPH = 25024
_NT = 16
_KB = 128
_D = 3
_CHJ = 6
_NJB = 66
_EPT = _NJB * _CHJ * _KB
_STRIPE = _NPH // _NT
_BR = 1000
_NB = _N // _BR
_HB = _NH // _BR
_DT = 1.0 / 9.0
_LN_EPS = 1e-5


@functools.cache
def _sc_edge_scatter():
    mesh = plsc.VectorSubcoreMesh(core_axis_name="c", subcore_axis_name="s")

    @functools.partial(
        pl.kernel,
        out_type=jax.ShapeDtypeStruct((2, _NPH, _DH), jnp.bfloat16),
        mesh=mesh,
        scratch_types=[
            pltpu.VMEM((2, _CHJ, _KB), jnp.int32),
            pltpu.VMEM((2, _CHJ, _KB), jnp.int32),
            pltpu.VMEM((_D, _KB, _DH), jnp.bfloat16),
            pltpu.VMEM_SHARED((_NPH, _DH), jnp.bfloat16),
            [pltpu.SemaphoreType.DMA] * _D,
            [pltpu.SemaphoreType.DMA] * _D,
            [pltpu.SemaphoreType.DMA] * 2,
        ],
        compiler_params=pltpu.CompilerParams(use_tc_tiling_on_sc=False),
    )
    def k(yh, srcA, srcB, dstA, dstB, out, src_v, dst_v, rows_v, acc,
          gsem, ssem, isem):
        c = lax.axis_index("c")
        t = lax.axis_index("s")
        z32 = jnp.zeros((32,), jnp.bfloat16)

        def zb(j, carry):
            for o in range(0, _DH, 32):
                rows_v[0, j, pl.ds(o, 32)] = z32
            return carry

        def do_pass(srch, dsth, q):
            def gather_start(slot, b, row):
                pltpu.async_copy(
                    yh.at[src_v.at[b, row]], rows_v.at[slot], gsem[slot])

            def gather_wait(slot):
                pltpu.make_async_copy(
                    yh.at[src_v.at[0, 0]], rows_v.at[slot],
                    gsem[slot]).wait()

            def scatter_start(slot, b, row):
                pltpu.async_copy(
                    rows_v.at[slot], acc.at[dst_v.at[b, row]], ssem[slot],
                    add=True)

            def scatter_wait(slot):
                pltpu.make_async_copy(
                    rows_v.at[slot], acc.at[dst_v.at[0, 0]],
                    ssem[slot]).wait()

            lax.fori_loop(0, _KB, zb, 0)

            def zc(i, carry):
                pltpu.sync_copy(
                    rows_v.at[0],
                    acc.at[pl.ds(t * _STRIPE + i * _KB, _KB)])
                return carry

            lax.fori_loop(0, _STRIPE // _KB, zc, 0)
            rem = _STRIPE % _KB
            if rem:
                pltpu.sync_copy(
                    rows_v.at[0, pl.ds(0, rem)],
                    acc.at[pl.ds(t * _STRIPE + _STRIPE - rem, rem)])
            plsc.subcore_barrier()

            pltpu.sync_copy(srch.at[t, pl.ds(0, _CHJ)], src_v.at[0])
            pltpu.sync_copy(dsth.at[t, pl.ds(0, _CHJ)], dst_v.at[0])

            def blk(jj, carry):
                b = jnp.bitwise_and(jj, 1)
                pb = 1 - b

                @pl.when(jj > 0)
                def _():
                    pltpu.make_async_copy(
                        srch.at[t, pl.ds(0, _CHJ)], src_v.at[0],
                        isem[0]).wait()
                    pltpu.make_async_copy(
                        dsth.at[t, pl.ds(0, _CHJ)], dst_v.at[0],
                        isem[1]).wait()

                for jb in range(_CHJ):
                    slot = jb % _D
                    if jb >= _D:
                        scatter_wait(slot)
                    else:
                        @pl.when(jj > 0)
                        def _():
                            scatter_wait(slot)
                    gather_start(slot, b, jb)
                    s2 = (jb - 2) % _D
                    if jb >= 2:
                        gather_wait(s2)
                        scatter_start(s2, b, jb - 2)
                    else:
                        @pl.when(jj > 0)
                        def _():
                            gather_wait(s2)
                            scatter_start(s2, pb, jb + _CHJ - 2)
                    if jb == 3:
                        @pl.when(jj < _NJB - 1)
                        def _():
                            pltpu.async_copy(
                                srch.at[t, pl.ds((jj + 1) * _CHJ, _CHJ)],
                                src_v.at[pb], isem[0])
                            pltpu.async_copy(
                                dsth.at[t, pl.ds((jj + 1) * _CHJ, _CHJ)],
                                dst_v.at[pb], isem[1])
                return carry

            lax.fori_loop(0, _NJB, blk, 0)
            lb = (_NJB - 1) % 2
            gather_wait((_CHJ - 2) % _D)
            scatter_start((_CHJ - 2) % _D, lb, _CHJ - 2)
            gather_wait((_CHJ - 1) % _D)
            scatter_start((_CHJ - 1) % _D, lb, _CHJ - 1)
            for slot in range(_D):
                scatter_wait(slot)
            plsc.subcore_barrier()
            pltpu.sync_copy(acc.at[pl.ds(t * _STRIPE, _STRIPE)],
                            out.at[q, pl.ds(t * _STRIPE, _STRIPE)])
            plsc.subcore_barrier()

        @pl.when(c == 0)
        def _():
            do_pass(srcA, dstA, 0)

        @pl.when(c == 1)
        def _():
            do_pass(srcB, dstB, 1)

    return k


_HI = jax.lax.Precision.HIGHEST


def _dinv_from_ones(so_blk):
    deg = so_blk[0, :, 0:1].astype(jnp.float32) + 1.0
    return lax.rsqrt(jnp.maximum(deg, 1e-12))


def _s_spec():
    return pl.BlockSpec((1, _BR, _DH), lambda r: (r // _HB, r % _HB, 0))


def _tc_init_body(x_ref, win_ref, bin_ref, wg_ref, so_ref, h_ref, y_ref):
    dinv = _dinv_from_ones(so_ref[...])
    h = jnp.dot(x_ref[...], win_ref[...], precision=_HI,
                preferred_element_type=jnp.float32) + bin_ref[...]
    h_ref[...] = h
    y = dinv * jnp.dot(h, wg_ref[...], precision=_HI,
                       preferred_element_type=jnp.float32)
    y_ref[...] = y.astype(jnp.bfloat16)


@functools.cache
def _tc_init():
    row = lambda r: (r, 0)
    fixed = lambda r: (0, 0)
    return pl.pallas_call(
        _tc_init_body,
        grid=(_NB,),
        in_specs=[
            pl.BlockSpec((_BR, _DF), row),
            pl.BlockSpec((_DF, _DH), fixed),
            pl.BlockSpec((1, _DH), fixed),
            pl.BlockSpec((_DH, _DH), fixed),
            _s_spec(),
        ],
        out_specs=[
            pl.BlockSpec((_BR, _DH), row),
            pl.BlockSpec((_BR, _DH), row),
        ],
        out_shape=[
            jax.ShapeDtypeStruct((_N, _DH), jnp.float32),
            jax.ShapeDtypeStruct((_N, _DH), jnp.bfloat16),
        ],
    )


def _tc_step_body(last, h_ref, y_ref, s_ref, so_ref,
                  wg_ref, bg_ref, g_ref, b_ref, *outs):
    dinv = _dinv_from_ones(so_ref[...])
    z = s_ref[0].astype(jnp.float32) + y_ref[...].astype(jnp.float32)
    pre = dinv * z + bg_ref[...]
    mu = jnp.mean(pre, axis=-1, keepdims=True)
    d = pre - mu
    var = jnp.mean(d * d, axis=-1, keepdims=True)
    dh = jnp.tanh(d * lax.rsqrt(var + _LN_EPS) * g_ref[...] + b_ref[...])
    h_new = h_ref[...] + _DT * dh
    outs[0][...] = h_new
    if not last:
        y = dinv * jnp.dot(h_new, wg_ref[...], precision=_HI,
                           preferred_element_type=jnp.float32)
        outs[1][...] = y.astype(jnp.bfloat16)


@functools.cache
def _tc_step(last):
    row = lambda r: (r, 0)
    fixed = lambda r: (0, 0)
    n_y_out = 0 if last else 1
    return pl.pallas_call(
        functools.partial(_tc_step_body, last),
        grid=(_NB,),
        in_specs=[
            pl.BlockSpec((_BR, _DH), row),
            pl.BlockSpec((_BR, _DH), row),
            _s_spec(),
            _s_spec(),
            pl.BlockSpec((_DH, _DH), fixed),
            pl.BlockSpec((1, _DH), fixed),
            pl.BlockSpec((1, _DH), fixed),
            pl.BlockSpec((1, _DH), fixed),
        ],
        out_specs=[pl.BlockSpec((_BR, _DH), row)] +
                  [pl.BlockSpec((_BR, _DH), row)] * n_y_out,
        out_shape=[jax.ShapeDtypeStruct((_N, _DH), jnp.float32)] +
                  [jax.ShapeDtypeStruct((_N, _DH), jnp.bfloat16)] * n_y_out,
    )


def _tc_final_body(h_ref, wout_ref, bout_ref, out_ref, acc_ref):
    r = pl.program_id(0)

    @pl.when(r == 0)
    def _():
        acc_ref[...] = jnp.zeros((8, _DH), jnp.float32)

    part = jnp.sum(h_ref[...], axis=0, keepdims=True)
    acc_ref[...] = acc_ref[...] + jnp.broadcast_to(part, (8, _DH))

    @pl.when(r == _NB - 1)
    def _():
        m = acc_ref[...] * (1.0 / _N)
        out_ref[...] = jnp.dot(m, wout_ref[...], precision=_HI,
                               preferred_element_type=jnp.float32) + \
            bout_ref[...]


@functools.cache
def _tc_final():
    fixed = lambda r: (0, 0)
    return pl.pallas_call(
        _tc_final_body,
        grid=(_NB,),
        in_specs=[
            pl.BlockSpec((_BR, _DH), lambda r: (r, 0)),
            pl.BlockSpec((_DH, _DH), fixed),
            pl.BlockSpec((1, _DH), fixed),
        ],
        out_specs=pl.BlockSpec((8, _DH), fixed),
        out_shape=jax.ShapeDtypeStruct((8, _DH), jnp.float32),
        scratch_shapes=[pltpu.VMEM((8, _DH), jnp.float32)],
    )


def kernel(x, edge_index, W_in, b_in, W_gcn, b_gcn, ln_g, ln_b, W_out, b_out):
    src = edge_index[0]
    dst = edge_index[1]
    pad = _NT * _EPT - _E
    srcp = jnp.concatenate([src, jnp.zeros((pad,), src.dtype)])
    dstp = jnp.concatenate([dst, jnp.full((pad,), _N, dst.dtype)])
    in_a = dstp < _NH
    shape = (_NT, _NJB * _CHJ, _KB)
    dummy = _NH + (jnp.arange(dstp.shape[0], dtype=dstp.dtype) % (_NPH - _NH))
    srcA = srcp.reshape(shape)
    srcB = srcp.reshape(shape)
    dstA = jnp.where(in_a, dstp, dummy).reshape(shape)
    dstB = jnp.where(in_a, dummy, dstp - _NH).reshape(shape)

    sc = _sc_edge_scatter()
    ones_tab = jnp.ones((_N, _DH), jnp.bfloat16)
    s_ones = sc(ones_tab, srcA, srcB, dstA, dstB)

    b_in2 = b_in.reshape(1, _DH)
    b_gcn2 = b_gcn.reshape(1, _DH)
    ln_g2 = ln_g.reshape(1, _DH)
    ln_b2 = ln_b.reshape(1, _DH)
    b_out2 = b_out.reshape(1, _DH)

    h, y = _tc_init()(x, W_in, b_in2, W_gcn, s_ones)
    for i in range(1, 10):
        s = sc(y, srcA, srcB, dstA, dstB)
        last = i == 9
        outs = _tc_step(last)(h, y, s, s_ones, W_gcn, b_gcn2, ln_g2, ln_b2)
        if last:
            (h,) = outs
        else:
            h, y = outs

    res = _tc_final()(h, W_out, b_out2)
    return res[0:1]


# speedup vs baseline: 32.7328x; 1.7429x over previous
import functools

import jax
import jax.numpy as jnp
from jax import lax
from jax.experimental import pallas as pl
from jax.experimental.pallas import tpu as pltpu
from jax.experimental.pallas import tpu_sc as plsc

_N = 50000
_E = 800000
_DF = 64
_DH = 128
_CW = 64
_NCH = 2
_NP = 50048
_NT = 16
_KB = 128
_CHJ = 8
_NJB = 49
_EPT = _NJB * _CHJ * _KB
_STRIPE = _NP // _NT
_BR = 1000
_NB = _N // _BR
_DT = 1.0 / 9.0
_LN_EPS = 1e-5


@functools.cache
def _sc_edge_scatter():
    mesh = plsc.VectorSubcoreMesh(core_axis_name="c", subcore_axis_name="s")

    @functools.partial(
        pl.kernel,
        out_type=jax.ShapeDtypeStruct((_NCH, _NP, _CW), jnp.bfloat16),
        mesh=mesh,
        scratch_types=[
            pltpu.VMEM((2, _CHJ, _KB), jnp.int32),
            pltpu.VMEM((2, _CHJ, _KB), jnp.int32),
            pltpu.VMEM((4, _KB, _CW), jnp.bfloat16),
            pltpu.VMEM_SHARED((_NP, _CW), jnp.bfloat16),
            [pltpu.SemaphoreType.DMA] * 4,
            [pltpu.SemaphoreType.DMA] * 4,
            [pltpu.SemaphoreType.DMA] * 2,
        ],
        compiler_params=pltpu.CompilerParams(use_tc_tiling_on_sc=False),
    )
    def k(y0h, y1h, srch, dsth, out, src_v, dst_v, rows_v, acc,
          gsem, ssem, isem):
        c = lax.axis_index("c")
        t = lax.axis_index("s")
        z32 = jnp.zeros((32,), jnp.bfloat16)

        def zb(j, carry):
            rows_v[0, j, pl.ds(0, 32)] = z32
            rows_v[0, j, pl.ds(32, 32)] = z32
            return carry

        def do_pass(yh, q):
            def gather_start(slot, b, row):
                pltpu.async_copy(
                    yh.at[src_v.at[b, row]], rows_v.at[slot], gsem[slot])

            def gather_wait(slot):
                pltpu.make_async_copy(
                    yh.at[src_v.at[0, 0]], rows_v.at[slot],
                    gsem[slot]).wait()

            def scatter_start(slot, b, row):
                pltpu.async_copy(
                    rows_v.at[slot], acc.at[dst_v.at[b, row]], ssem[slot],
                    add=True)

            def scatter_wait(slot):
                pltpu.make_async_copy(
                    rows_v.at[slot], acc.at[dst_v.at[0, 0]],
                    ssem[slot]).wait()

            lax.fori_loop(0, _KB, zb, 0)

            def zc(i, carry):
                pltpu.sync_copy(
                    rows_v.at[0],
                    acc.at[pl.ds(t * _STRIPE + i * _KB, _KB)])
                return carry

            lax.fori_loop(0, _STRIPE // _KB, zc, 0)
            rem = _STRIPE % _KB
            if rem:
                pltpu.sync_copy(
                    rows_v.at[0, pl.ds(0, rem)],
                    acc.at[pl.ds(t * _STRIPE + _STRIPE - rem, rem)])
            plsc.subcore_barrier()

            pltpu.sync_copy(srch.at[t, pl.ds(0, _CHJ)], src_v.at[0])
            pltpu.sync_copy(dsth.at[t, pl.ds(0, _CHJ)], dst_v.at[0])

            def blk(jj, carry):
                b = jnp.bitwise_and(jj, 1)
                pb = 1 - b

                @pl.when(jj > 0)
                def _():
                    pltpu.make_async_copy(
                        srch.at[t, pl.ds(0, _CHJ)], src_v.at[0],
                        isem[0]).wait()
                    pltpu.make_async_copy(
                        dsth.at[t, pl.ds(0, _CHJ)], dst_v.at[0],
                        isem[1]).wait()

                for jb in range(_CHJ):
                    slot = jb % 4
                    if jb >= 4:
                        scatter_wait(slot)
                    else:
                        @pl.when(jj > 0)
                        def _():
                            scatter_wait(slot)
                    gather_start(slot, b, jb)
                    s2 = (jb - 2) % 4
                    if jb >= 2:
                        gather_wait(s2)
                        scatter_start(s2, b, jb - 2)
                    else:
                        @pl.when(jj > 0)
                        def _():
                            gather_wait(s2)
                            scatter_start(s2, pb, jb + 6)
                    if jb == 4:
                        @pl.when(jj < _NJB - 1)
                        def _():
                            pltpu.async_copy(
                                srch.at[t, pl.ds((jj + 1) * _CHJ, _CHJ)],
                                src_v.at[pb], isem[0])
                            pltpu.async_copy(
                                dsth.at[t, pl.ds((jj + 1) * _CHJ, _CHJ)],
                                dst_v.at[pb], isem[1])
                return carry

            lax.fori_loop(0, _NJB, blk, 0)
            lb = (_NJB - 1) % 2
            gather_wait(2)
            scatter_start(2, lb, 6)
            gather_wait(3)
            scatter_start(3, lb, 7)
            for slot in range(4):
                scatter_wait(slot)
            plsc.subcore_barrier()
            pltpu.sync_copy(acc.at[pl.ds(t * _STRIPE, _STRIPE)],
                            out.at[q, pl.ds(t * _STRIPE, _STRIPE)])
            plsc.subcore_barrier()

        @pl.when(c == 0)
        def _():
            do_pass(y0h, 0)

        @pl.when(c == 1)
        def _():
            do_pass(y1h, 1)

    return k


def _sc_call(y_full, srcT, dstT):
    y0 = lax.slice(y_full, (0, 0), (_N, _CW))
    y1 = lax.slice(y_full, (0, _CW), (_N, _DH))
    s = _sc_edge_scatter()(y0, y1, srcT, dstT)
    return jnp.concatenate([s[0], s[1]], axis=-1)


_HI = jax.lax.Precision.HIGHEST


def _dinv_from_ones(so_blk):
    deg = so_blk[:, 0:1].astype(jnp.float32) + 1.0
    return lax.rsqrt(jnp.maximum(deg, 1e-12))


def _tc_init_body(x_ref, win_ref, bin_ref, wg_ref, so_ref, h_ref, y_ref):
    dinv = _dinv_from_ones(so_ref[...])
    h = jnp.dot(x_ref[...], win_ref[...], precision=_HI,
                preferred_element_type=jnp.float32) + bin_ref[...]
    h_ref[...] = h
    y = dinv * jnp.dot(h, wg_ref[...], precision=_HI,
                       preferred_element_type=jnp.float32)
    y_ref[...] = y.astype(jnp.bfloat16)


@functools.cache
def _tc_init():
    row = lambda r: (r, 0)
    fixed = lambda r: (0, 0)
    return pl.pallas_call(
        _tc_init_body,
        grid=(_NB,),
        in_specs=[
            pl.BlockSpec((_BR, _DF), row),
            pl.BlockSpec((_DF, _DH), fixed),
            pl.BlockSpec((1, _DH), fixed),
            pl.BlockSpec((_DH, _DH), fixed),
            pl.BlockSpec((_BR, _DH), row),
        ],
        out_specs=[
            pl.BlockSpec((_BR, _DH), row),
            pl.BlockSpec((_BR, _DH), row),
        ],
        out_shape=[
            jax.ShapeDtypeStruct((_N, _DH), jnp.float32),
            jax.ShapeDtypeStruct((_N, _DH), jnp.bfloat16),
        ],
    )


def _tc_step_body(last, h_ref, y_ref, s_ref, so_ref,
                  wg_ref, bg_ref, g_ref, b_ref, *outs):
    dinv = _dinv_from_ones(so_ref[...])
    z = s_ref[...].astype(jnp.float32) + y_ref[...].astype(jnp.float32)
    pre = dinv * z + bg_ref[...]
    mu = jnp.mean(pre, axis=-1, keepdims=True)
    d = pre - mu
    var = jnp.mean(d * d, axis=-1, keepdims=True)
    dh = jnp.tanh(d * lax.rsqrt(var + _LN_EPS) * g_ref[...] + b_ref[...])
    h_new = h_ref[...] + _DT * dh
    outs[0][...] = h_new
    if not last:
        y = dinv * jnp.dot(h_new, wg_ref[...], precision=_HI,
                           preferred_element_type=jnp.float32)
        outs[1][...] = y.astype(jnp.bfloat16)


@functools.cache
def _tc_step(last):
    row = lambda r: (r, 0)
    fixed = lambda r: (0, 0)
    n_y_out = 0 if last else 1
    return pl.pallas_call(
        functools.partial(_tc_step_body, last),
        grid=(_NB,),
        in_specs=[
            pl.BlockSpec((_BR, _DH), row),
            pl.BlockSpec((_BR, _DH), row),
            pl.BlockSpec((_BR, _DH), row),
            pl.BlockSpec((_BR, _DH), row),
            pl.BlockSpec((_DH, _DH), fixed),
            pl.BlockSpec((1, _DH), fixed),
            pl.BlockSpec((1, _DH), fixed),
            pl.BlockSpec((1, _DH), fixed),
        ],
        out_specs=[pl.BlockSpec((_BR, _DH), row)] +
                  [pl.BlockSpec((_BR, _DH), row)] * n_y_out,
        out_shape=[jax.ShapeDtypeStruct((_N, _DH), jnp.float32)] +
                  [jax.ShapeDtypeStruct((_N, _DH), jnp.bfloat16)] * n_y_out,
    )


def _tc_final_body(h_ref, wout_ref, bout_ref, out_ref, acc_ref):
    r = pl.program_id(0)

    @pl.when(r == 0)
    def _():
        acc_ref[...] = jnp.zeros((8, _DH), jnp.float32)

    part = jnp.sum(h_ref[...], axis=0, keepdims=True)
    acc_ref[...] = acc_ref[...] + jnp.broadcast_to(part, (8, _DH))

    @pl.when(r == _NB - 1)
    def _():
        m = acc_ref[...] * (1.0 / _N)
        out_ref[...] = jnp.dot(m, wout_ref[...], precision=_HI,
                               preferred_element_type=jnp.float32) + \
            bout_ref[...]


@functools.cache
def _tc_final():
    fixed = lambda r: (0, 0)
    return pl.pallas_call(
        _tc_final_body,
        grid=(_NB,),
        in_specs=[
            pl.BlockSpec((_BR, _DH), lambda r: (r, 0)),
            pl.BlockSpec((_DH, _DH), fixed),
            pl.BlockSpec((1, _DH), fixed),
        ],
        out_specs=pl.BlockSpec((8, _DH), fixed),
        out_shape=jax.ShapeDtypeStruct((8, _DH), jnp.float32),
        scratch_shapes=[pltpu.VMEM((8, _DH), jnp.float32)],
    )


def kernel(x, edge_index, W_in, b_in, W_gcn, b_gcn, ln_g, ln_b, W_out, b_out):
    src = edge_index[0]
    dst = edge_index[1]
    pad = _NT * _EPT - _E
    srcT = jnp.concatenate(
        [src, jnp.zeros((pad,), src.dtype)]).reshape(_NT, _NJB * _CHJ, _KB)
    dstT = jnp.concatenate(
        [dst, jnp.full((pad,), _NP - 1, dst.dtype)]).reshape(
            _NT, _NJB * _CHJ, _KB)

    ones_full = jnp.ones((_N, _DH), jnp.bfloat16)
    s_ones = _sc_call(ones_full, srcT, dstT)[: _N]

    b_in2 = b_in.reshape(1, _DH)
    b_gcn2 = b_gcn.reshape(1, _DH)
    ln_g2 = ln_g.reshape(1, _DH)
    ln_b2 = ln_b.reshape(1, _DH)
    b_out2 = b_out.reshape(1, _DH)

    h, y = _tc_init()(x, W_in, b_in2, W_gcn, s_ones)
    for i in range(1, 10):
        s = _sc_call(y, srcT, dstT)[: _N]
        last = i == 9
        outs = _tc_step(last)(h, y, s, s_ones, W_gcn, b_gcn2, ln_g2, ln_b2)
        if last:
            (h,) = outs
        else:
            h, y = outs

    res = _tc_final()(h, W_out, b_out2)
    return res[0:1]


# speedup vs baseline: 35.8533x; 1.0953x over previous
import functools

import jax
import jax.numpy as jnp
from jax import lax
from jax.experimental import pallas as pl
from jax.experimental.pallas import tpu as pltpu
from jax.experimental.pallas import tpu_sc as plsc

_N = 50000
_E = 800000
_DF = 64
_DH = 128
_CW = 64
_NCH = 2
_NP = 50048
_NT = 16
_KB = 128
_CHJ = 8
_NJB = 49
_EPT = _NJB * _CHJ * _KB
_STRIPE = _NP // _NT
_BR = 1000
_NB = _N // _BR
_DT = 1.0 / 9.0
_LN_EPS = 1e-5


@functools.cache
def _sc_edge_scatter():
    mesh = plsc.VectorSubcoreMesh(core_axis_name="c", subcore_axis_name="s")

    @functools.partial(
        pl.kernel,
        out_type=jax.ShapeDtypeStruct((_NP, _DH), jnp.bfloat16),
        mesh=mesh,
        scratch_types=[
            pltpu.VMEM((2, _CHJ, _KB), jnp.int32),
            pltpu.VMEM((2, _CHJ, _KB), jnp.int32),
            pltpu.VMEM((4, _KB, _CW), jnp.bfloat16),
            pltpu.VMEM_SHARED((_NP, _CW), jnp.bfloat16),
            [pltpu.SemaphoreType.DMA] * 4,
            [pltpu.SemaphoreType.DMA] * 4,
            [pltpu.SemaphoreType.DMA] * 2,
        ],
        compiler_params=pltpu.CompilerParams(use_tc_tiling_on_sc=False),
    )
    def k(y0h, y1h, srch, dsth, out, src_v, dst_v, rows_v, acc,
          gsem, ssem, isem):
        c = lax.axis_index("c")
        t = lax.axis_index("s")
        z32 = jnp.zeros((32,), jnp.bfloat16)

        def zb(j, carry):
            rows_v[0, j, pl.ds(0, 32)] = z32
            rows_v[0, j, pl.ds(32, 32)] = z32
            return carry

        def do_pass(yh, q):
            def gather_start(slot, b, row):
                pltpu.async_copy(
                    yh.at[src_v.at[b, row]], rows_v.at[slot], gsem[slot])

            def gather_wait(slot):
                pltpu.make_async_copy(
                    yh.at[src_v.at[0, 0]], rows_v.at[slot],
                    gsem[slot]).wait()

            def scatter_start(slot, b, row):
                pltpu.async_copy(
                    rows_v.at[slot], acc.at[dst_v.at[b, row]], ssem[slot],
                    add=True)

            def scatter_wait(slot):
                pltpu.make_async_copy(
                    rows_v.at[slot], acc.at[dst_v.at[0, 0]],
                    ssem[slot]).wait()

            lax.fori_loop(0, _KB, zb, 0)

            def zc(i, carry):
                pltpu.sync_copy(
                    rows_v.at[0],
                    acc.at[pl.ds(t * _STRIPE + i * _KB, _KB)])
                return carry

            lax.fori_loop(0, _STRIPE // _KB, zc, 0)
            rem = _STRIPE % _KB
            if rem:
                pltpu.sync_copy(
                    rows_v.at[0, pl.ds(0, rem)],
                    acc.at[pl.ds(t * _STRIPE + _STRIPE - rem, rem)])
            plsc.subcore_barrier()

            pltpu.sync_copy(srch.at[t, pl.ds(0, _CHJ)], src_v.at[0])
            pltpu.sync_copy(dsth.at[t, pl.ds(0, _CHJ)], dst_v.at[0])

            def blk(jj, carry):
                b = jnp.bitwise_and(jj, 1)
                pb = 1 - b

                @pl.when(jj > 0)
                def _():
                    pltpu.make_async_copy(
                        srch.at[t, pl.ds(0, _CHJ)], src_v.at[0],
                        isem[0]).wait()
                    pltpu.make_async_copy(
                        dsth.at[t, pl.ds(0, _CHJ)], dst_v.at[0],
                        isem[1]).wait()

                for jb in range(_CHJ):
                    slot = jb % 4
                    if jb >= 4:
                        scatter_wait(slot)
                    else:
                        @pl.when(jj > 0)
                        def _():
                            scatter_wait(slot)
                    gather_start(slot, b, jb)
                    s2 = (jb - 2) % 4
                    if jb >= 2:
                        gather_wait(s2)
                        scatter_start(s2, b, jb - 2)
                    else:
                        @pl.when(jj > 0)
                        def _():
                            gather_wait(s2)
                            scatter_start(s2, pb, jb + 6)
                    if jb == 4:
                        @pl.when(jj < _NJB - 1)
                        def _():
                            pltpu.async_copy(
                                srch.at[t, pl.ds((jj + 1) * _CHJ, _CHJ)],
                                src_v.at[pb], isem[0])
                            pltpu.async_copy(
                                dsth.at[t, pl.ds((jj + 1) * _CHJ, _CHJ)],
                                dst_v.at[pb], isem[1])
                return carry

            lax.fori_loop(0, _NJB, blk, 0)
            lb = (_NJB - 1) % 2
            gather_wait(2)
            scatter_start(2, lb, 6)
            gather_wait(3)
            scatter_start(3, lb, 7)
            for slot in range(4):
                scatter_wait(slot)
            plsc.subcore_barrier()
            pltpu.sync_copy(acc.at[pl.ds(t * _STRIPE, _STRIPE)],
                            out.at[pl.ds(t * _STRIPE, _STRIPE),
                                   pl.ds(q * _CW, _CW)])
            plsc.subcore_barrier()

        @pl.when(c == 0)
        def _():
            do_pass(y0h, 0)

        @pl.when(c == 1)
        def _():
            do_pass(y1h, 1)

    return k


def _sc_call(y_full, srcT, dstT):
    y0 = lax.slice(y_full, (0, 0), (_N, _CW))
    y1 = lax.slice(y_full, (0, _CW), (_N, _DH))
    return _sc_edge_scatter()(y0, y1, srcT, dstT)


_HI = jax.lax.Precision.HIGHEST


def _dinv_from_ones(so_blk):
    deg = so_blk[:, 0:1].astype(jnp.float32) + 1.0
    return lax.rsqrt(jnp.maximum(deg, 1e-12))


def _tc_init_body(x_ref, win_ref, bin_ref, wg_ref, so_ref, h_ref, y_ref):
    dinv = _dinv_from_ones(so_ref[...])
    h = jnp.dot(x_ref[...], win_ref[...], precision=_HI,
                preferred_element_type=jnp.float32) + bin_ref[...]
    h_ref[...] = h
    y = dinv * jnp.dot(h, wg_ref[...], precision=_HI,
                       preferred_element_type=jnp.float32)
    y_ref[...] = y.astype(jnp.bfloat16)


@functools.cache
def _tc_init():
    row = lambda r: (r, 0)
    fixed = lambda r: (0, 0)
    return pl.pallas_call(
        _tc_init_body,
        grid=(_NB,),
        in_specs=[
            pl.BlockSpec((_BR, _DF), row),
            pl.BlockSpec((_DF, _DH), fixed),
            pl.BlockSpec((1, _DH), fixed),
            pl.BlockSpec((_DH, _DH), fixed),
            pl.BlockSpec((_BR, _DH), row),
        ],
        out_specs=[
            pl.BlockSpec((_BR, _DH), row),
            pl.BlockSpec((_BR, _DH), row),
        ],
        out_shape=[
            jax.ShapeDtypeStruct((_N, _DH), jnp.float32),
            jax.ShapeDtypeStruct((_N, _DH), jnp.bfloat16),
        ],
    )


def _tc_step_body(last, h_ref, y_ref, s_ref, so_ref,
                  wg_ref, bg_ref, g_ref, b_ref, *outs):
    dinv = _dinv_from_ones(so_ref[...])
    z = s_ref[...].astype(jnp.float32) + y_ref[...].astype(jnp.float32)
    pre = dinv * z + bg_ref[...]
    mu = jnp.mean(pre, axis=-1, keepdims=True)
    d = pre - mu
    var = jnp.mean(d * d, axis=-1, keepdims=True)
    dh = jnp.tanh(d * lax.rsqrt(var + _LN_EPS) * g_ref[...] + b_ref[...])
    h_new = h_ref[...] + _DT * dh
    outs[0][...] = h_new
    if not last:
        y = dinv * jnp.dot(h_new, wg_ref[...], precision=_HI,
                           preferred_element_type=jnp.float32)
        outs[1][...] = y.astype(jnp.bfloat16)


@functools.cache
def _tc_step(last):
    row = lambda r: (r, 0)
    fixed = lambda r: (0, 0)
    n_y_out = 0 if last else 1
    return pl.pallas_call(
        functools.partial(_tc_step_body, last),
        grid=(_NB,),
        in_specs=[
            pl.BlockSpec((_BR, _DH), row),
            pl.BlockSpec((_BR, _DH), row),
            pl.BlockSpec((_BR, _DH), row),
            pl.BlockSpec((_BR, _DH), row),
            pl.BlockSpec((_DH, _DH), fixed),
            pl.BlockSpec((1, _DH), fixed),
            pl.BlockSpec((1, _DH), fixed),
            pl.BlockSpec((1, _DH), fixed),
        ],
        out_specs=[pl.BlockSpec((_BR, _DH), row)] +
                  [pl.BlockSpec((_BR, _DH), row)] * n_y_out,
        out_shape=[jax.ShapeDtypeStruct((_N, _DH), jnp.float32)] +
                  [jax.ShapeDtypeStruct((_N, _DH), jnp.bfloat16)] * n_y_out,
    )


def _tc_final_body(h_ref, wout_ref, bout_ref, out_ref, acc_ref):
    r = pl.program_id(0)

    @pl.when(r == 0)
    def _():
        acc_ref[...] = jnp.zeros((8, _DH), jnp.float32)

    part = jnp.sum(h_ref[...], axis=0, keepdims=True)
    acc_ref[...] = acc_ref[...] + jnp.broadcast_to(part, (8, _DH))

    @pl.when(r == _NB - 1)
    def _():
        m = acc_ref[...] * (1.0 / _N)
        out_ref[...] = jnp.dot(m, wout_ref[...], precision=_HI,
                               preferred_element_type=jnp.float32) + \
            bout_ref[...]


@functools.cache
def _tc_final():
    fixed = lambda r: (0, 0)
    return pl.pallas_call(
        _tc_final_body,
        grid=(_NB,),
        in_specs=[
            pl.BlockSpec((_BR, _DH), lambda r: (r, 0)),
            pl.BlockSpec((_DH, _DH), fixed),
            pl.BlockSpec((1, _DH), fixed),
        ],
        out_specs=pl.BlockSpec((8, _DH), fixed),
        out_shape=jax.ShapeDtypeStruct((8, _DH), jnp.float32),
        scratch_shapes=[pltpu.VMEM((8, _DH), jnp.float32)],
    )


def kernel(x, edge_index, W_in, b_in, W_gcn, b_gcn, ln_g, ln_b, W_out, b_out):
    src = edge_index[0]
    dst = edge_index[1]
    pad = _NT * _EPT - _E
    srcT = jnp.concatenate(
        [src, jnp.zeros((pad,), src.dtype)]).reshape(_NT, _NJB * _CHJ, _KB)
    dstT = jnp.concatenate(
        [dst, jnp.full((pad,), _NP - 1, dst.dtype)]).reshape(
            _NT, _NJB * _CHJ, _KB)

    ones_full = jnp.ones((_N, _DH), jnp.bfloat16)
    s_ones = _sc_call(ones_full, srcT, dstT)

    b_in2 = b_in.reshape(1, _DH)
    b_gcn2 = b_gcn.reshape(1, _DH)
    ln_g2 = ln_g.reshape(1, _DH)
    ln_b2 = ln_b.reshape(1, _DH)
    b_out2 = b_out.reshape(1, _DH)

    h, y = _tc_init()(x, W_in, b_in2, W_gcn, s_ones)
    for i in range(1, 10):
        s = _sc_call(y, srcT, dstT)
        last = i == 9
        outs = _tc_step(last)(h, y, s, s_ones, W_gcn, b_gcn2, ln_g2, ln_b2)
        if last:
            (h,) = outs
        else:
            h, y = outs

    res = _tc_final()(h, W_out, b_out2)
    return res[0:1]
